# Initial kernel scaffold; baseline (speedup 1.0000x reference)
import jax
import jax.numpy as jnp
from jax.experimental import pallas as pl


def kernel(x, edge_index, W1, b1, W2, b2, W_skip, b_skip, W_score):
    raise NotImplementedError("write your pallas kernel here")


# speedup vs baseline: 1.2205x; 1.2205x over previous
import functools

import jax
import jax.numpy as jnp
from jax.experimental import pallas as pl
from jax.experimental.pallas import tpu as pltpu

MBLK = 1000


def _mm_kernel(x_ref, w_ref, o_ref):
    o_ref[...] = jnp.dot(x_ref[...], w_ref[...],
                         preferred_element_type=jnp.float32)


def _matmul(x, w, blk_m):
    M, K = x.shape
    _, N = w.shape
    return pl.pallas_call(
        _mm_kernel,
        grid=(M // blk_m,),
        in_specs=[
            pl.BlockSpec((blk_m, K), lambda i: (i, 0)),
            pl.BlockSpec((K, N), lambda i: (0, 0)),
        ],
        out_specs=pl.BlockSpec((blk_m, N), lambda i: (i, 0)),
        out_shape=jax.ShapeDtypeStruct((M, N), jnp.float32),
    )(x, w)


def _gate_kernel(x1_ref, ws_ref, raw_ref, x1g_ref):
    x1 = x1_ref[...]
    raw = jnp.dot(x1, ws_ref[...], preferred_element_type=jnp.float32)
    raw_ref[...] = raw
    gate = jnp.tanh(raw[:, 0:1])
    x1g_ref[...] = x1 * gate


def _final_kernel(x1_ref, w_ref, b_ref, up_ref, o_ref):
    o_ref[...] = (jnp.dot(x1_ref[...], w_ref[...],
                          preferred_element_type=jnp.float32)
                  + b_ref[...] + up_ref[...])


def kernel(x, edge_index, W1, b1, W2, b2, W_skip, b_skip, W_score):
    n = x.shape[0]
    in_dim = x.shape[1]
    hid = W1.shape[1]
    out_dim = W2.shape[1]
    K = 1000
    row = edge_index[0]
    col = edge_index[1]
    E = row.shape[0]

    y = _matmul(x, W1, MBLK)
    ones = jnp.ones((E,), jnp.float32)
    deg = jax.ops.segment_sum(ones, col, num_segments=n) + 1.0
    dinv = deg ** -0.5
    w_e = dinv[row] * dinv[col]
    z = jax.ops.segment_sum(w_e[:, None] * y[row], col, num_segments=n)
    x1 = jax.nn.relu(z + (dinv * dinv)[:, None] * y + b1)

    Wsc = jnp.pad(W_score, ((0, 0), (0, 127)))
    raw2, x1g = pl.pallas_call(
        _gate_kernel,
        grid=(n // MBLK,),
        in_specs=[
            pl.BlockSpec((MBLK, hid), lambda i: (i, 0)),
            pl.BlockSpec((hid, 128), lambda i: (0, 0)),
        ],
        out_specs=[
            pl.BlockSpec((MBLK, 128), lambda i: (i, 0)),
            pl.BlockSpec((MBLK, hid), lambda i: (i, 0)),
        ],
        out_shape=[
            jax.ShapeDtypeStruct((n, 128), jnp.float32),
            jax.ShapeDtypeStruct((n, hid), jnp.float32),
        ],
    )(x1, Wsc)
    raw = raw2[:, 0]

    _, kept = jax.lax.top_k(raw, K)
    keep_mask = jnp.zeros((n,), dtype=bool).at[kept].set(True)
    degb = jnp.bincount(row, length=n).astype(jnp.int32)
    src = jnp.concatenate([row, col])
    dst = jnp.concatenate([col, row])
    enc = jnp.where(keep_mask[dst], degb[dst] * n + (n - 1 - dst), -1)
    best_enc = jax.ops.segment_max(enc, src, num_segments=n)
    has_cand = best_enc >= 0
    cluster_kept = jnp.full((n,), -1, dtype=jnp.int32).at[kept].set(
        jnp.arange(K, dtype=jnp.int32))
    best_global_kept = kept[jnp.argmax(degb[kept])]
    w_best = jnp.where(has_cand, (n - 1) - (jnp.maximum(best_enc, 0) % n),
                       best_global_kept)
    cluster_id = jnp.where(keep_mask, cluster_kept, cluster_kept[w_best])

    sums = jax.ops.segment_sum(x1g, cluster_id, num_segments=K)
    counts = jnp.maximum(jnp.bincount(cluster_id, length=K), 1).astype(
        jnp.float32)[:, None]
    x_p = sums / counts

    cu = cluster_id[row]
    cv = cluster_id[col]
    code = cu * K + cv
    uniq, wcnt = jnp.unique(code, return_counts=True, size=E, fill_value=0)
    uc = uniq // K
    vc = uniq % K
    m = uc != vc
    ew_p = jnp.where(m, wcnt, 0).astype(jnp.float32)

    xw2 = _matmul(x_p, W2, K)
    deg2 = jax.ops.segment_sum(ew_p, vc, num_segments=K) + 1.0
    dinv2 = deg2 ** -0.5
    w2e = dinv2[uc] * ew_p * dinv2[vc]
    z2 = jax.ops.segment_sum(w2e[:, None] * xw2[uc], vc, num_segments=K)
    x_p2 = z2 + (dinv2 * dinv2)[:, None] * xw2 + b2

    up = x_p2[cluster_id]
    out = pl.pallas_call(
        _final_kernel,
        grid=(n // MBLK,),
        in_specs=[
            pl.BlockSpec((MBLK, hid), lambda i: (i, 0)),
            pl.BlockSpec((hid, out_dim), lambda i: (0, 0)),
            pl.BlockSpec((1, out_dim), lambda i: (0, 0)),
            pl.BlockSpec((MBLK, out_dim), lambda i: (i, 0)),
        ],
        out_specs=pl.BlockSpec((MBLK, out_dim), lambda i: (i, 0)),
        out_shape=jax.ShapeDtypeStruct((n, out_dim), jnp.float32),
    )(x1, W_skip, b_skip.reshape(1, -1), up)
    return (out, jnp.asarray(0.0, dtype=jnp.float32))


# speedup vs baseline: 1.6078x; 1.3172x over previous
import functools

import jax
import jax.numpy as jnp
from jax.experimental import pallas as pl
from jax.experimental.pallas import tpu as pltpu

MBLK = 1000


def _mm_kernel(x_ref, w_ref, o_ref):
    o_ref[...] = jnp.dot(x_ref[...], w_ref[...],
                         preferred_element_type=jnp.float32)


def _matmul(x, w, blk_m):
    M, K = x.shape
    _, N = w.shape
    return pl.pallas_call(
        _mm_kernel,
        grid=(M // blk_m,),
        in_specs=[
            pl.BlockSpec((blk_m, K), lambda i: (i, 0)),
            pl.BlockSpec((K, N), lambda i: (0, 0)),
        ],
        out_specs=pl.BlockSpec((blk_m, N), lambda i: (i, 0)),
        out_shape=jax.ShapeDtypeStruct((M, N), jnp.float32),
    )(x, w)


def _gate_kernel(x1_ref, ws_ref, raw_ref, x1g_ref):
    x1 = x1_ref[...]
    raw = jnp.dot(x1, ws_ref[...], preferred_element_type=jnp.float32)
    raw_ref[...] = raw
    gate = jnp.tanh(raw[:, 0:1])
    x1g_ref[...] = x1 * gate


def _pool_kernel(cid_ref, x1g_ref, sums_ref, cnt_ref):
    i = pl.program_id(0)
    cid2 = cid_ref[0]
    K = sums_ref.shape[0]
    ohT = (jax.lax.broadcasted_iota(jnp.int32, (K, MBLK), 0)
           == cid2).astype(jnp.float32)
    contrib = jnp.dot(ohT, x1g_ref[...], preferred_element_type=jnp.float32)
    cntc = jnp.dot(ohT, jnp.ones((MBLK, 128), jnp.float32),
                   preferred_element_type=jnp.float32)

    @pl.when(i == 0)
    def _init():
        sums_ref[...] = contrib
        cnt_ref[...] = cntc

    @pl.when(i != 0)
    def _acc():
        sums_ref[...] += contrib
        cnt_ref[...] += cntc


def _conv2_kernel(a_ref, sums_ref, cnt_ref, w2_ref, b2_ref, o_ref):
    K = a_ref.shape[0]
    A = a_ref[...]
    ior = jax.lax.broadcasted_iota(jnp.int32, (K, K), 0)
    ioc = jax.lax.broadcasted_iota(jnp.int32, (K, K), 1)
    A = jnp.where(ior == ioc, 0.0, A)
    ones128 = jnp.ones((K, 128), jnp.float32)
    deg2 = jax.lax.dot_general(
        A, ones128, (((0,), (0,)), ((), ())),
        preferred_element_type=jnp.float32)[:, 0:1] + 1.0
    dinv2 = jax.lax.rsqrt(deg2)
    cnt = jnp.maximum(cnt_ref[...][:, 0:1], 1.0)
    x_p = sums_ref[...] / cnt
    xw2 = jnp.dot(x_p, w2_ref[...], preferred_element_type=jnp.float32)
    g = dinv2 * xw2
    z2 = dinv2 * jax.lax.dot_general(
        A, g, (((0,), (0,)), ((), ())), preferred_element_type=jnp.float32)
    o_ref[...] = z2 + (dinv2 * dinv2) * xw2 + b2_ref[...]


def _final_kernel(x1_ref, w_ref, b_ref, cid_ref, xp2_ref, o_ref):
    K = xp2_ref.shape[0]
    cid2 = cid_ref[0]
    ohT = (jax.lax.broadcasted_iota(jnp.int32, (K, MBLK), 0)
           == cid2).astype(jnp.float32)
    up = jax.lax.dot_general(
        ohT, xp2_ref[...], (((0,), (0,)), ((), ())),
        preferred_element_type=jnp.float32)
    o_ref[...] = (jnp.dot(x1_ref[...], w_ref[...],
                          preferred_element_type=jnp.float32)
                  + b_ref[...] + up)


def kernel(x, edge_index, W1, b1, W2, b2, W_skip, b_skip, W_score):
    n = x.shape[0]
    in_dim = x.shape[1]
    hid = W1.shape[1]
    out_dim = W2.shape[1]
    K = 1000
    row = edge_index[0]
    col = edge_index[1]
    E = row.shape[0]

    y = _matmul(x, W1, MBLK)
    ones = jnp.ones((E,), jnp.float32)
    deg = jax.ops.segment_sum(ones, col, num_segments=n) + 1.0
    dinv = deg ** -0.5
    w_e = dinv[row] * dinv[col]
    z = jax.ops.segment_sum(w_e[:, None] * y[row], col, num_segments=n)
    x1 = jax.nn.relu(z + (dinv * dinv)[:, None] * y + b1)

    Wsc = jnp.pad(W_score, ((0, 0), (0, 127)))
    raw2, x1g = pl.pallas_call(
        _gate_kernel,
        grid=(n // MBLK,),
        in_specs=[
            pl.BlockSpec((MBLK, hid), lambda i: (i, 0)),
            pl.BlockSpec((hid, 128), lambda i: (0, 0)),
        ],
        out_specs=[
            pl.BlockSpec((MBLK, 128), lambda i: (i, 0)),
            pl.BlockSpec((MBLK, hid), lambda i: (i, 0)),
        ],
        out_shape=[
            jax.ShapeDtypeStruct((n, 128), jnp.float32),
            jax.ShapeDtypeStruct((n, hid), jnp.float32),
        ],
    )(x1, Wsc)
    raw = raw2[:, 0]

    _, kept = jax.lax.top_k(raw, K)
    keep_mask = jnp.zeros((n,), dtype=bool).at[kept].set(True)
    degb = jnp.bincount(row, length=n).astype(jnp.int32)
    src = jnp.concatenate([row, col])
    dst = jnp.concatenate([col, row])
    enc = jnp.where(keep_mask[dst], degb[dst] * n + (n - 1 - dst), -1)
    best_enc = jax.ops.segment_max(enc, src, num_segments=n)
    has_cand = best_enc >= 0
    cluster_kept = jnp.full((n,), -1, dtype=jnp.int32).at[kept].set(
        jnp.arange(K, dtype=jnp.int32))
    best_global_kept = kept[jnp.argmax(degb[kept])]
    w_best = jnp.where(has_cand, (n - 1) - (jnp.maximum(best_enc, 0) % n),
                       best_global_kept)
    cluster_id = jnp.where(keep_mask, cluster_kept, cluster_kept[w_best])

    cid_blocks = cluster_id.reshape(n // MBLK, 1, MBLK)
    sums, cnt = pl.pallas_call(
        _pool_kernel,
        grid=(n // MBLK,),
        in_specs=[
            pl.BlockSpec((1, 1, MBLK), lambda i: (i, 0, 0)),
            pl.BlockSpec((MBLK, hid), lambda i: (i, 0)),
        ],
        out_specs=[
            pl.BlockSpec((K, hid), lambda i: (0, 0)),
            pl.BlockSpec((K, 128), lambda i: (0, 0)),
        ],
        out_shape=[
            jax.ShapeDtypeStruct((K, hid), jnp.float32),
            jax.ShapeDtypeStruct((K, 128), jnp.float32),
        ],
    )(cid_blocks, x1g)

    code = cluster_id[row] * K + cluster_id[col]
    A = jnp.zeros((K * K,), jnp.float32).at[code].add(1.0).reshape(K, K)
    x_p2 = pl.pallas_call(
        _conv2_kernel,
        in_specs=[
            pl.BlockSpec((K, K), lambda: (0, 0)),
            pl.BlockSpec((K, hid), lambda: (0, 0)),
            pl.BlockSpec((K, 128), lambda: (0, 0)),
            pl.BlockSpec((hid, out_dim), lambda: (0, 0)),
            pl.BlockSpec((1, out_dim), lambda: (0, 0)),
        ],
        out_specs=pl.BlockSpec((K, out_dim), lambda: (0, 0)),
        out_shape=jax.ShapeDtypeStruct((K, out_dim), jnp.float32),
    )(A, sums, cnt, W2, b2.reshape(1, -1))

    out = pl.pallas_call(
        _final_kernel,
        grid=(n // MBLK,),
        in_specs=[
            pl.BlockSpec((MBLK, hid), lambda i: (i, 0)),
            pl.BlockSpec((hid, out_dim), lambda i: (0, 0)),
            pl.BlockSpec((1, out_dim), lambda i: (0, 0)),
            pl.BlockSpec((1, 1, MBLK), lambda i: (i, 0, 0)),
            pl.BlockSpec((K, out_dim), lambda i: (0, 0)),
        ],
        out_specs=pl.BlockSpec((MBLK, out_dim), lambda i: (i, 0)),
        out_shape=jax.ShapeDtypeStruct((n, out_dim), jnp.float32),
    )(x1, W_skip, b_skip.reshape(1, -1), cid_blocks, x_p2)
    return (out, jnp.asarray(0.0, dtype=jnp.float32))


# speedup vs baseline: 2.0885x; 1.2990x over previous
import functools

import jax
import jax.numpy as jnp
from jax.experimental import pallas as pl
from jax.experimental.pallas import tpu as pltpu

MBLK = 1000


def _mm_kernel(x_ref, w_ref, o_ref):
    o_ref[...] = jnp.dot(x_ref[...], w_ref[...],
                         preferred_element_type=jnp.float32)


def _matmul(x, w, blk_m):
    M, K = x.shape
    _, N = w.shape
    return pl.pallas_call(
        _mm_kernel,
        grid=(M // blk_m,),
        in_specs=[
            pl.BlockSpec((blk_m, K), lambda i: (i, 0)),
            pl.BlockSpec((K, N), lambda i: (0, 0)),
        ],
        out_specs=pl.BlockSpec((blk_m, N), lambda i: (i, 0)),
        out_shape=jax.ShapeDtypeStruct((M, N), jnp.float32),
    )(x, w)


def _conv1_mm_kernel(x_ref, w_ref, dinv_ref, o_ref):
    o_ref[...] = jnp.dot(x_ref[...], w_ref[...],
                         preferred_element_type=jnp.float32) * dinv_ref[...]


def _cg_kernel(z_ref, y2_ref, dinv_ref, b1_ref, ws_ref,
               x1_ref, raw_ref, x1g_ref):
    x1 = jax.nn.relu((z_ref[...] + y2_ref[...]) * dinv_ref[...] + b1_ref[...])
    x1_ref[...] = x1
    raw = jnp.dot(x1, ws_ref[...], preferred_element_type=jnp.float32)
    raw_ref[...] = raw
    gate = jnp.tanh(raw[:, 0:1])
    x1g_ref[...] = x1 * gate


def _pool_kernel(cid_ref, x1g_ref, sums_ref, cnt_ref):
    i = pl.program_id(0)
    cid2 = cid_ref[0]
    K = sums_ref.shape[0]
    ohT = (jax.lax.broadcasted_iota(jnp.int32, (K, MBLK), 0)
           == cid2).astype(jnp.float32)
    contrib = jnp.dot(ohT, x1g_ref[...], preferred_element_type=jnp.float32)
    cntc = jnp.dot(ohT, jnp.ones((MBLK, 128), jnp.float32),
                   preferred_element_type=jnp.float32)

    @pl.when(i == 0)
    def _init():
        sums_ref[...] = contrib
        cnt_ref[...] = cntc

    @pl.when(i != 0)
    def _acc():
        sums_ref[...] += contrib
        cnt_ref[...] += cntc


def _conv2_kernel(a_ref, sums_ref, cnt_ref, w2_ref, b2_ref, o_ref):
    K = a_ref.shape[0]
    A = a_ref[...]
    ior = jax.lax.broadcasted_iota(jnp.int32, (K, K), 0)
    ioc = jax.lax.broadcasted_iota(jnp.int32, (K, K), 1)
    A = jnp.where(ior == ioc, 0.0, A)
    ones128 = jnp.ones((K, 128), jnp.float32)
    deg2 = jax.lax.dot_general(
        A, ones128, (((0,), (0,)), ((), ())),
        preferred_element_type=jnp.float32)[:, 0:1] + 1.0
    dinv2 = jax.lax.rsqrt(deg2)
    cnt = jnp.maximum(cnt_ref[...][:, 0:1], 1.0)
    x_p = sums_ref[...] / cnt
    xw2 = jnp.dot(x_p, w2_ref[...], preferred_element_type=jnp.float32)
    g = dinv2 * xw2
    z2 = dinv2 * jax.lax.dot_general(
        A, g, (((0,), (0,)), ((), ())), preferred_element_type=jnp.float32)
    o_ref[...] = z2 + (dinv2 * dinv2) * xw2 + b2_ref[...]


def _final_kernel(x1_ref, w_ref, b_ref, cid_ref, xp2_ref, o_ref):
    K = xp2_ref.shape[0]
    cid2 = cid_ref[0]
    ohT = (jax.lax.broadcasted_iota(jnp.int32, (K, MBLK), 0)
           == cid2).astype(jnp.float32)
    up = jax.lax.dot_general(
        ohT, xp2_ref[...], (((0,), (0,)), ((), ())),
        preferred_element_type=jnp.float32)
    o_ref[...] = (jnp.dot(x1_ref[...], w_ref[...],
                          preferred_element_type=jnp.float32)
                  + b_ref[...] + up)


def kernel(x, edge_index, W1, b1, W2, b2, W_skip, b_skip, W_score):
    n = x.shape[0]
    in_dim = x.shape[1]
    hid = W1.shape[1]
    out_dim = W2.shape[1]
    K = 1000
    row = edge_index[0]
    col = edge_index[1]
    E = row.shape[0]

    ones = jnp.ones((E,), jnp.float32)
    deg = jax.ops.segment_sum(ones, col, num_segments=n) + 1.0
    dinv = deg ** -0.5
    dinv_col = dinv[:, None]
    y2 = pl.pallas_call(
        _conv1_mm_kernel,
        grid=(n // MBLK,),
        in_specs=[
            pl.BlockSpec((MBLK, in_dim), lambda i: (i, 0)),
            pl.BlockSpec((in_dim, hid), lambda i: (0, 0)),
            pl.BlockSpec((MBLK, 1), lambda i: (i, 0)),
        ],
        out_specs=pl.BlockSpec((MBLK, hid), lambda i: (i, 0)),
        out_shape=jax.ShapeDtypeStruct((n, hid), jnp.float32),
    )(x, W1, dinv_col)
    z_raw = jax.ops.segment_sum(y2[row], col, num_segments=n)

    Wsc = jnp.pad(W_score, ((0, 0), (0, 127)))
    x1, raw2, x1g = pl.pallas_call(
        _cg_kernel,
        grid=(n // MBLK,),
        in_specs=[
            pl.BlockSpec((MBLK, hid), lambda i: (i, 0)),
            pl.BlockSpec((MBLK, hid), lambda i: (i, 0)),
            pl.BlockSpec((MBLK, 1), lambda i: (i, 0)),
            pl.BlockSpec((1, hid), lambda i: (0, 0)),
            pl.BlockSpec((hid, 128), lambda i: (0, 0)),
        ],
        out_specs=[
            pl.BlockSpec((MBLK, hid), lambda i: (i, 0)),
            pl.BlockSpec((MBLK, 128), lambda i: (i, 0)),
            pl.BlockSpec((MBLK, hid), lambda i: (i, 0)),
        ],
        out_shape=[
            jax.ShapeDtypeStruct((n, hid), jnp.float32),
            jax.ShapeDtypeStruct((n, 128), jnp.float32),
            jax.ShapeDtypeStruct((n, hid), jnp.float32),
        ],
    )(z_raw, y2, dinv_col, b1.reshape(1, -1), Wsc)
    raw = raw2[:, 0]

    _, kept = jax.lax.top_k(raw, K)
    keep_mask = jnp.zeros((n,), dtype=bool).at[kept].set(True)
    degb = jnp.bincount(row, length=n).astype(jnp.int32)
    src = jnp.concatenate([row, col])
    dst = jnp.concatenate([col, row])
    enc = jnp.where(keep_mask[dst], degb[dst] * n + (n - 1 - dst), -1)
    best_enc = jax.ops.segment_max(enc, src, num_segments=n)
    has_cand = best_enc >= 0
    cluster_kept = jnp.full((n,), -1, dtype=jnp.int32).at[kept].set(
        jnp.arange(K, dtype=jnp.int32))
    best_global_kept = kept[jnp.argmax(degb[kept])]
    w_best = jnp.where(has_cand, (n - 1) - (jnp.maximum(best_enc, 0) % n),
                       best_global_kept)
    cluster_id = jnp.where(keep_mask, cluster_kept, cluster_kept[w_best])

    cid_blocks = cluster_id.reshape(n // MBLK, 1, MBLK)
    sums, cnt = pl.pallas_call(
        _pool_kernel,
        grid=(n // MBLK,),
        in_specs=[
            pl.BlockSpec((1, 1, MBLK), lambda i: (i, 0, 0)),
            pl.BlockSpec((MBLK, hid), lambda i: (i, 0)),
        ],
        out_specs=[
            pl.BlockSpec((K, hid), lambda i: (0, 0)),
            pl.BlockSpec((K, 128), lambda i: (0, 0)),
        ],
        out_shape=[
            jax.ShapeDtypeStruct((K, hid), jnp.float32),
            jax.ShapeDtypeStruct((K, 128), jnp.float32),
        ],
    )(cid_blocks, x1g)

    code = cluster_id[row] * K + cluster_id[col]
    A = jnp.zeros((K * K,), jnp.float32).at[code].add(1.0).reshape(K, K)
    x_p2 = pl.pallas_call(
        _conv2_kernel,
        in_specs=[
            pl.BlockSpec((K, K), lambda: (0, 0)),
            pl.BlockSpec((K, hid), lambda: (0, 0)),
            pl.BlockSpec((K, 128), lambda: (0, 0)),
            pl.BlockSpec((hid, out_dim), lambda: (0, 0)),
            pl.BlockSpec((1, out_dim), lambda: (0, 0)),
        ],
        out_specs=pl.BlockSpec((K, out_dim), lambda: (0, 0)),
        out_shape=jax.ShapeDtypeStruct((K, out_dim), jnp.float32),
    )(A, sums, cnt, W2, b2.reshape(1, -1))

    out = pl.pallas_call(
        _final_kernel,
        grid=(n // MBLK,),
        in_specs=[
            pl.BlockSpec((MBLK, hid), lambda i: (i, 0)),
            pl.BlockSpec((hid, out_dim), lambda i: (0, 0)),
            pl.BlockSpec((1, out_dim), lambda i: (0, 0)),
            pl.BlockSpec((1, 1, MBLK), lambda i: (i, 0, 0)),
            pl.BlockSpec((K, out_dim), lambda i: (0, 0)),
        ],
        out_specs=pl.BlockSpec((MBLK, out_dim), lambda i: (i, 0)),
        out_shape=jax.ShapeDtypeStruct((n, out_dim), jnp.float32),
    )(x1, W_skip, b_skip.reshape(1, -1), cid_blocks, x_p2)
    return (out, jnp.asarray(0.0, dtype=jnp.float32))
